# baseline (speedup 1.0000x reference)
import functools

import jax
import jax.numpy as jnp
from jax import lax
from jax.experimental import pallas as pl
from jax.experimental.pallas import tpu as pltpu
from jax.experimental.pallas import tpu_sc as plsc

N_USERS = 100000
N_ITEMS = 50000
FEAT = 128
EMBED = 64

NC = 2
NS = 16
NW = NC * NS

CHUNK = 128
N_FULL = N_USERS // CHUNK
TAIL = N_USERS - N_FULL * CHUNK
N_CHUNKS = N_FULL + 1
K_PER_W = (N_CHUNKS + NW - 1) // NW

@functools.lru_cache(maxsize=1)
def _make_user_gather():
    mesh = plsc.VectorSubcoreMesh(core_axis_name="c", subcore_axis_name="s")

    @functools.partial(
        pl.kernel,
        out_type=jax.ShapeDtypeStruct((N_USERS, EMBED), jnp.float32),
        mesh=mesh,
        scratch_types=[
            pltpu.VMEM((CHUNK,), jnp.int32),
            pltpu.VMEM((CHUNK, EMBED), jnp.float32),
            pltpu.SemaphoreType.DMA,
        ],
        compiler_params=pltpu.CompilerParams(use_tc_tiling_on_sc=False),
    )
    def _user_gather(idx_hbm, table_hbm, out_hbm, idx_v, rows_v, sem):
        wid = lax.axis_index("s") * NC + lax.axis_index("c")

        def body(k, _):
            c = wid + k * NW

            @pl.when(c < N_CHUNKS)
            def _():
                pltpu.sync_copy(idx_hbm.at[c], idx_v)
                pltpu.async_copy(table_hbm.at[idx_v], rows_v, sem).wait()

                @pl.when(c < N_FULL)
                def _():
                    pltpu.sync_copy(
                        rows_v, out_hbm.at[pl.ds(c * CHUNK, CHUNK)]
                    )

                @pl.when(c == N_FULL)
                def _():
                    pltpu.sync_copy(
                        rows_v.at[pl.ds(0, TAIL)],
                        out_hbm.at[pl.ds(N_FULL * CHUNK, TAIL)],
                    )

            return None

        return lax.fori_loop(0, K_PER_W, body, None)

    return _user_gather


def _mm_body(x_ref, w_ref, b_ref, o_ref):
    o_ref[...] = (
        jnp.dot(x_ref[...], w_ref[...], preferred_element_type=jnp.float32)
        + b_ref[...]
    )


_ROWS_BLK = 2000
_item_linear = pl.pallas_call(
    _mm_body,
    grid=(N_ITEMS // _ROWS_BLK,),
    in_specs=[
        pl.BlockSpec((_ROWS_BLK, FEAT), lambda i: (i, 0)),
        pl.BlockSpec((FEAT, EMBED), lambda i: (0, 0)),
        pl.BlockSpec((1, EMBED), lambda i: (0, 0)),
    ],
    out_specs=pl.BlockSpec((_ROWS_BLK, EMBED), lambda i: (i, 0)),
    out_shape=jax.ShapeDtypeStruct((N_ITEMS, EMBED), jnp.float32),
    compiler_params=pltpu.CompilerParams(
        dimension_semantics=("parallel",),
    ),
)


def kernel(user_ids, item_features, user_table, item_W, item_b):
    ids = user_ids.astype(jnp.int32)
    ids_pad = jnp.pad(ids, (0, N_CHUNKS * CHUNK - N_USERS)).reshape(
        N_CHUNKS, CHUNK
    )
    h_user = _make_user_gather()(ids_pad, user_table)
    h_item = _item_linear(item_features, item_W, item_b.reshape(1, EMBED))
    return (h_user, h_item)
